# baseline (speedup 1.0000x reference)
import functools

import jax
import jax.numpy as jnp
from jax import lax
from jax.experimental import pallas as pl
from jax.experimental.pallas import tpu as pltpu
from jax.experimental.pallas import tpu_sc as plsc

HIDDEN = 128
EMB = 64
ROWS = 300

NC = 2
NS = 16
NW = NC * NS

B_TOTAL = 16384 * 200
IDX_ROWS = B_TOTAL // 128
ROWS_PER_W = IDX_ROWS // NW
CH_ROWS = 4
N_ITERS = ROWS_PER_W // CH_ROWS


def _mm_body(t_ref, w_ref, b_ref, o_ref):
    o_ref[...] = (
        jnp.dot(t_ref[...], w_ref[...], preferred_element_type=jnp.float32)
        + b_ref[...]
    )


def _fused_table(table, W, b):
    return pl.pallas_call(
        _mm_body,
        out_shape=jax.ShapeDtypeStruct((ROWS, HIDDEN), jnp.float32),
    )(table, W, b.reshape(1, HIDDEN))


def _gather_body(idx_hbm, tab_hbm, out_hbm, idx_v, rows_v, sem):
    c = lax.axis_index("c")
    s = lax.axis_index("s")
    wid = s * NC + c
    row0 = wid * ROWS_PER_W

    def step(i, carry):
        r = row0 + i * CH_ROWS
        pltpu.sync_copy(idx_hbm.at[pl.ds(r, CH_ROWS)], idx_v)
        for j in range(CH_ROWS):
            for m in range(8):
                sl = (j, pl.ds(m * 16, 16))
                idx_v[sl] = jnp.clip(idx_v[sl], 30, 300) - 30
        copies = [
            pltpu.async_copy(
                tab_hbm.at[idx_v.at[j]],
                rows_v.at[pl.ds(j * 128, 128)],
                sem,
            )
            for j in range(CH_ROWS)
        ]
        for cp in copies:
            cp.wait()
        pltpu.sync_copy(rows_v, out_hbm.at[pl.ds(r * 128, CH_ROWS * 128)])
        return carry

    lax.fori_loop(0, N_ITERS, step, 0)


@functools.partial(jax.jit, static_argnums=())
def _gather(idx2d, fused):
    mesh = plsc.VectorSubcoreMesh(core_axis_name="c", subcore_axis_name="s")
    return pl.kernel(
        _gather_body,
        out_type=jax.ShapeDtypeStruct((B_TOTAL, HIDDEN), jnp.float32),
        mesh=mesh,
        scratch_types=[
            pltpu.VMEM((CH_ROWS, 128), jnp.int32),
            pltpu.VMEM((CH_ROWS * 128, HIDDEN), jnp.float32),
            pltpu.SemaphoreType.DMA,
        ],
    )(idx2d, fused)


def kernel(tempo, table, W, b):
    fused = _fused_table(table, W, b)
    idx2d = tempo.astype(jnp.int32).reshape(IDX_ROWS, 128)
    out = _gather(idx2d, fused)
    return out.reshape(16384, 200, HIDDEN)


# speedup vs baseline: 20.4624x; 20.4624x over previous
import functools

import jax
import jax.numpy as jnp
from jax import lax
from jax.experimental import pallas as pl
from jax.experimental.pallas import tpu as pltpu
from jax.experimental.pallas import tpu_sc as plsc

HIDDEN = 128
EMB = 64
ROWS = 300

NC = 2
NS = 16
NW = NC * NS

B_TOTAL = 16384 * 200
IDX_ROWS = B_TOTAL // 128
ROWS_PER_W = IDX_ROWS // NW
BLK = 16
N_BLK = ROWS_PER_W // BLK
NSLOT = 4


def _mm_body(t_ref, w_ref, b_ref, o_ref):
    o_ref[...] = (
        jnp.dot(t_ref[...], w_ref[...], preferred_element_type=jnp.float32)
        + b_ref[...]
    )


def _fused_table(table, W, b):
    return pl.pallas_call(
        _mm_body,
        out_shape=jax.ShapeDtypeStruct((ROWS, HIDDEN), jnp.float32),
    )(table, W, b.reshape(1, HIDDEN))


def _gather_body(idx_hbm, tab_hbm, out_hbm, tab_v, idx_v, rows_v,
                 w0, w1, w2, w3):
    wsems = (w0, w1, w2, w3)
    c = lax.axis_index("c")
    s = lax.axis_index("s")
    wid = s * NC + c
    row0 = wid * ROWS_PER_W

    @pl.when(s == 0)
    def _():
        pltpu.sync_copy(tab_hbm, tab_v)
    plsc.subcore_barrier()

    def out_desc(g, slot):
        return pltpu.make_async_copy(
            rows_v.at[slot],
            out_hbm.at[pl.ds((row0 + g) * 128, 128)],
            wsems[slot],
        )

    def block(i, carry):
        pltpu.sync_copy(idx_hbm.at[pl.ds(row0 + i * BLK, BLK)], idx_v)
        for q in range(BLK):
            g = i * BLK + q
            slot = q % NSLOT
            if q >= NSLOT:
                out_desc(g - NSLOT, slot).wait()
            else:
                @pl.when(i > 0)
                def _():
                    out_desc(g - NSLOT, slot).wait()
            for m in range(8):
                sl = (q, pl.ds(m * 16, 16))
                idx_v[sl] = jnp.clip(idx_v[sl], 30, 300) - 30
            pltpu.sync_copy(tab_v.at[idx_v.at[q]], rows_v.at[slot])
            out_desc(g, slot).start()
        return carry

    lax.fori_loop(0, N_BLK, block, 0)
    last = N_BLK * BLK
    for q in range(NSLOT):
        g = last - NSLOT + q
        out_desc(g, g % NSLOT).wait()


@functools.partial(jax.jit, static_argnums=())
def _gather(idx2d, fused):
    mesh = plsc.VectorSubcoreMesh(core_axis_name="c", subcore_axis_name="s")
    return pl.kernel(
        _gather_body,
        out_type=jax.ShapeDtypeStruct((B_TOTAL, HIDDEN), jnp.float32),
        mesh=mesh,
        scratch_types=[
            pltpu.VMEM_SHARED((ROWS, HIDDEN), jnp.float32),
            pltpu.VMEM((BLK, 128), jnp.int32),
            pltpu.VMEM((NSLOT, 128, HIDDEN), jnp.float32),
            pltpu.SemaphoreType.DMA,
            pltpu.SemaphoreType.DMA,
            pltpu.SemaphoreType.DMA,
            pltpu.SemaphoreType.DMA,
        ],
    )(idx2d, fused)


def kernel(tempo, table, W, b):
    fused = _fused_table(table, W, b)
    idx2d = tempo.astype(jnp.int32).reshape(IDX_ROWS, 128)
    out = _gather(idx2d, fused)
    return out.reshape(16384, 200, HIDDEN)


# speedup vs baseline: 21.4821x; 1.0498x over previous
import functools

import jax
import jax.numpy as jnp
from jax import lax
from jax.experimental import pallas as pl
from jax.experimental.pallas import tpu as pltpu
from jax.experimental.pallas import tpu_sc as plsc

HIDDEN = 128
EMB = 64
ROWS = 300

NC = 2
NS = 16
NW = NC * NS

B_TOTAL = 16384 * 200
IDX_ROWS = B_TOTAL // 128
ROWS_PER_W = IDX_ROWS // NW
BLK = 16
N_BLK = ROWS_PER_W // BLK
NSLOT = 4


def _mm_body(t_ref, w_ref, b_ref, o_ref):
    o_ref[...] = (
        jnp.dot(t_ref[...], w_ref[...], preferred_element_type=jnp.float32)
        + b_ref[...]
    )


def _fused_table(table, W, b):
    return pl.pallas_call(
        _mm_body,
        out_shape=jax.ShapeDtypeStruct((ROWS, HIDDEN), jnp.float32),
    )(table, W, b.reshape(1, HIDDEN))


def _gather_body(idx_hbm, tab_hbm, out_hbm, tab_v, idx_v, rows_v,
                 g0, g1, w0, w1, w2, w3):
    gsems = (g0, g1)
    wsems = (w0, w1, w2, w3)
    c = lax.axis_index("c")
    s = lax.axis_index("s")
    wid = s * NC + c
    row0 = wid * ROWS_PER_W

    @pl.when(s == 0)
    def _():
        pltpu.sync_copy(tab_hbm, tab_v)
    plsc.subcore_barrier()

    def out_desc(g, slot):
        return pltpu.make_async_copy(
            rows_v.at[slot],
            out_hbm.at[pl.ds((row0 + g) * 128, 128)],
            wsems[slot],
        )

    def gat_desc(q, slot):
        return pltpu.make_async_copy(
            tab_v.at[idx_v.at[q]], rows_v.at[slot], gsems[q % 2]
        )

    def clip_and_fire(q, slot):
        for m in range(8):
            sl = (q, pl.ds(m * 16, 16))
            idx_v[sl] = jnp.clip(idx_v[sl], 30, 300) - 30
        gat_desc(q, slot).start()

    pltpu.sync_copy(idx_hbm.at[pl.ds(row0, BLK)], idx_v)
    clip_and_fire(0, 0)

    def block(i, carry):
        for q in range(BLK):
            g = i * BLK + q
            slot = q % NSLOT
            nslot = (q + 1) % NSLOT
            if q + 1 < BLK:
                if q + 1 >= NSLOT:
                    out_desc(g + 1 - NSLOT, nslot).wait()
                else:
                    @pl.when(i > 0)
                    def _():
                        out_desc(g + 1 - NSLOT, nslot).wait()
                clip_and_fire(q + 1, nslot)
            gat_desc(q, slot).wait()
            out_desc(g, slot).start()
        @pl.when(i + 1 < N_BLK)
        def _():
            pltpu.sync_copy(
                idx_hbm.at[pl.ds(row0 + (i + 1) * BLK, BLK)], idx_v
            )
            g1 = (i + 1) * BLK
            out_desc(g1 - NSLOT, 0).wait()
            clip_and_fire(0, 0)
        return carry

    lax.fori_loop(0, N_BLK, block, 0)
    last = N_BLK * BLK
    for d in range(NSLOT):
        g = last - NSLOT + d
        out_desc(g, g % NSLOT).wait()


@functools.partial(jax.jit, static_argnums=())
def _gather(idx2d, fused):
    mesh = plsc.VectorSubcoreMesh(core_axis_name="c", subcore_axis_name="s")
    return pl.kernel(
        _gather_body,
        out_type=jax.ShapeDtypeStruct((B_TOTAL, HIDDEN), jnp.float32),
        mesh=mesh,
        scratch_types=[
            pltpu.VMEM_SHARED((ROWS, HIDDEN), jnp.float32),
            pltpu.VMEM((BLK, 128), jnp.int32),
            pltpu.VMEM((NSLOT, 128, HIDDEN), jnp.float32),
            pltpu.SemaphoreType.DMA,
            pltpu.SemaphoreType.DMA,
            pltpu.SemaphoreType.DMA,
            pltpu.SemaphoreType.DMA,
            pltpu.SemaphoreType.DMA,
            pltpu.SemaphoreType.DMA,
        ],
    )(idx2d, fused)


def kernel(tempo, table, W, b):
    fused = _fused_table(table, W, b)
    idx2d = tempo.astype(jnp.int32).reshape(IDX_ROWS, 128)
    out = _gather(idx2d, fused)
    return out.reshape(16384, 200, HIDDEN)


# speedup vs baseline: 21.9874x; 1.0235x over previous
import functools

import jax
import jax.numpy as jnp
from jax import lax
from jax.experimental import pallas as pl
from jax.experimental.pallas import tpu as pltpu
from jax.experimental.pallas import tpu_sc as plsc

HIDDEN = 128
EMB = 64
ROWS = 300

NC = 2
NS = 16
NW = NC * NS

B_TOTAL = 16384 * 200
IDX_ROWS = B_TOTAL // 128
ROWS_PER_W = IDX_ROWS // NW
BLK = 40
N_BLK = ROWS_PER_W // BLK
NSLOT = 7


def _mm_body(t_ref, w_ref, b_ref, o_ref):
    o_ref[...] = (
        jnp.dot(t_ref[...], w_ref[...], preferred_element_type=jnp.float32)
        + b_ref[...]
    )


def _fused_table(table, W, b):
    return pl.pallas_call(
        _mm_body,
        out_shape=jax.ShapeDtypeStruct((ROWS, HIDDEN), jnp.float32),
    )(table, W, b.reshape(1, HIDDEN))


def _gather_body(idx_hbm, tab_hbm, out_hbm, tab_v, idx_v, rows_v,
                 g0, g1, wsem):
    gsems = (g0, g1)
    c = lax.axis_index("c")
    s = lax.axis_index("s")
    wid = s * NC + c
    row0 = wid * ROWS_PER_W

    @pl.when(s == 0)
    def _():
        pltpu.sync_copy(tab_hbm, tab_v)
    plsc.subcore_barrier()

    def out_desc(g):
        slot = lax.rem(g, NSLOT)
        return pltpu.make_async_copy(
            rows_v.at[slot],
            out_hbm.at[pl.ds((row0 + g) * 128, 128)],
            wsem.at[slot],
        )

    def gat_desc(q, g):
        slot = lax.rem(g, NSLOT)
        return pltpu.make_async_copy(
            tab_v.at[idx_v.at[q]], rows_v.at[slot], gsems[q % 2]
        )

    def clip_and_fire(q, g):
        for m in range(8):
            sl = (q, pl.ds(m * 16, 16))
            idx_v[sl] = jnp.clip(idx_v[sl], 30, 300) - 30
        gat_desc(q, g).start()

    pltpu.sync_copy(idx_hbm.at[pl.ds(row0, BLK)], idx_v)
    clip_and_fire(0, 0)

    def block(i, carry):
        for q in range(BLK):
            g = i * BLK + q
            if q + 1 < BLK:
                if q + 1 >= NSLOT:
                    out_desc(g + 1 - NSLOT).wait()
                else:
                    @pl.when(i > 0)
                    def _():
                        out_desc(g + 1 - NSLOT).wait()
                clip_and_fire(q + 1, g + 1)
            gat_desc(q, g).wait()
            out_desc(g).start()
        @pl.when(i + 1 < N_BLK)
        def _():
            pltpu.sync_copy(
                idx_hbm.at[pl.ds(row0 + (i + 1) * BLK, BLK)], idx_v
            )
            g1 = (i + 1) * BLK
            out_desc(g1 - NSLOT).wait()
            clip_and_fire(0, g1)
        return carry

    lax.fori_loop(0, N_BLK, block, 0)
    last = N_BLK * BLK
    for d in range(NSLOT):
        out_desc(last - NSLOT + d).wait()


@functools.partial(jax.jit, static_argnums=())
def _gather(idx2d, fused):
    mesh = plsc.VectorSubcoreMesh(core_axis_name="c", subcore_axis_name="s")
    return pl.kernel(
        _gather_body,
        out_type=jax.ShapeDtypeStruct((B_TOTAL, HIDDEN), jnp.float32),
        mesh=mesh,
        scratch_types=[
            pltpu.VMEM_SHARED((ROWS, HIDDEN), jnp.float32),
            pltpu.VMEM((BLK, 128), jnp.int32),
            pltpu.VMEM((NSLOT, 128, HIDDEN), jnp.float32),
            pltpu.SemaphoreType.DMA,
            pltpu.SemaphoreType.DMA,
            pltpu.SemaphoreType.DMA((NSLOT,)),
        ],
    )(idx2d, fused)


def kernel(tempo, table, W, b):
    fused = _fused_table(table, W, b)
    idx2d = tempo.astype(jnp.int32).reshape(IDX_ROWS, 128)
    out = _gather(idx2d, fused)
    return out.reshape(16384, 200, HIDDEN)
